# baseline (speedup 1.0000x reference)
import functools

import jax
import jax.numpy as jnp
from jax import lax
from jax.experimental import pallas as pl
from jax.experimental.pallas import tpu as pltpu
from jax.experimental.pallas import tpu_sc as plsc

_NC, _NS = 2, 16
_NW = _NC * _NS
_CROWS = 32
_NBUF = 2


@functools.lru_cache(maxsize=None)
def _build_gather(V, D, batch, hist, crows, nb):
    rows_per_w = batch // _NW
    n = rows_per_w // crows
    mesh = plsc.VectorSubcoreMesh(core_axis_name="c", subcore_axis_name="s")

    chunk = crows * hist
    scratch = ([pltpu.VMEM((crows, hist), jnp.int32)] * nb
               + [pltpu.VMEM((chunk, D), jnp.float32)] * nb
               + [pltpu.SemaphoreType.DMA] * (3 * nb))

    @functools.partial(
        pl.kernel,
        mesh=mesh,
        out_type=jax.ShapeDtypeStruct((batch * hist, D), jnp.float32),
        compiler_params=pltpu.CompilerParams(use_tc_tiling_on_sc=False),
        scratch_types=scratch,
    )
    def gather_k(table_hbm, idx_hbm, out_hbm, *bufs):
        idx_v = bufs[0:nb]
        rows_v = bufs[nb:2 * nb]
        si = bufs[2 * nb:3 * nb]
        sg = bufs[3 * nb:4 * nb]
        ss = bufs[4 * nb:5 * nb]

        wid = lax.axis_index("s") * _NC + lax.axis_index("c")
        base = wid * rows_per_w

        def idx_load(c, b):
            return pltpu.async_copy(
                idx_hbm.at[pl.ds(base + c * crows, crows), :], idx_v[b], si[b])

        def gather(b):
            return [
                pltpu.async_copy(
                    table_hbm.at[idx_v[b].at[r]],
                    rows_v[b].at[pl.ds(r * hist, hist)], sg[b])
                for r in range(crows)
            ]

        def store(c, b):
            return pltpu.async_copy(
                rows_v[b],
                out_hbm.at[pl.ds((base + c * crows) * hist, chunk)],
                ss[b])

        h_i = [None] * n
        h_g = [None] * n
        h_s = [None] * n

        for c in range(min(nb, n)):
            h_i[c] = idx_load(c, c)

        for c in range(n):
            b = c % nb
            if c >= nb:
                h_s[c - nb].wait()
            h_i[c].wait()
            h_g[c] = gather(b)
            d = c - (nb - 1)
            if d >= 0:
                for h in h_g[d]:
                    h.wait()
                h_s[d] = store(d, d % nb)
                if c + 1 < n:
                    h_i[c + 1] = idx_load(c + 1, (c + 1) % nb)

        for d in range(max(0, n - (nb - 1)), n):
            for h in h_g[d]:
                h.wait()
            h_s[d] = store(d, d % nb)
        for d in range(max(0, n - nb), n):
            h_s[d].wait()

    return gather_k


def kernel(inputs, embeddings):
    batch, hist = inputs.shape
    V, D = embeddings.shape
    if inputs.dtype != jnp.int32:
        inputs = inputs.astype(jnp.int32)
    out = _build_gather(V, D, batch, hist, _CROWS, _NBUF)(embeddings, inputs)
    return out.reshape(batch, hist, D)


# speedup vs baseline: 1.6292x; 1.6292x over previous
import functools

import jax
import jax.numpy as jnp
from jax import lax
from jax.experimental import pallas as pl
from jax.experimental.pallas import tpu as pltpu
from jax.experimental.pallas import tpu_sc as plsc

_NC, _NS = 2, 16
_NW = _NC * _NS
_CROWS = 32
_NBUF = 2


@functools.lru_cache(maxsize=None)
def _build_gather(V, D, batch, hist, crows, nb):
    rows_per_w = batch // _NW
    n = rows_per_w // crows
    mesh = plsc.VectorSubcoreMesh(core_axis_name="c", subcore_axis_name="s")

    scratch = ([pltpu.VMEM((crows, hist), jnp.int32)] * nb
               + [pltpu.VMEM((crows, hist, D), jnp.float32)] * nb
               + [pltpu.SemaphoreType.DMA] * (3 * nb))

    @functools.partial(
        pl.kernel,
        mesh=mesh,
        out_type=jax.ShapeDtypeStruct((batch, hist, D), jnp.float32),
        compiler_params=pltpu.CompilerParams(use_tc_tiling_on_sc=False),
        scratch_types=scratch,
    )
    def gather_k(table_hbm, idx_hbm, out_hbm, *bufs):
        idx_v = bufs[0:nb]
        rows_v = bufs[nb:2 * nb]
        si = bufs[2 * nb:3 * nb]
        sg = bufs[3 * nb:4 * nb]
        ss = bufs[4 * nb:5 * nb]

        wid = lax.axis_index("s") * _NC + lax.axis_index("c")
        base = wid * rows_per_w

        def idx_load(c, b):
            return pltpu.async_copy(
                idx_hbm.at[pl.ds(base + c * crows, crows), :], idx_v[b], si[b])

        def gather(b):
            return [
                pltpu.async_copy(
                    table_hbm.at[idx_v[b].at[r]], rows_v[b].at[r], sg[b])
                for r in range(crows)
            ]

        def store(c, b):
            return pltpu.async_copy(
                rows_v[b], out_hbm.at[pl.ds(base + c * crows, crows), :, :],
                ss[b])

        h_i = [None] * n
        h_g = [None] * n
        h_s = [None] * n

        for c in range(min(nb, n)):
            h_i[c] = idx_load(c, c)

        for c in range(n):
            b = c % nb
            if c >= nb:
                h_s[c - nb].wait()
            h_i[c].wait()
            h_g[c] = gather(b)
            d = c - (nb - 1)
            if d >= 0:
                for h in h_g[d]:
                    h.wait()
                h_s[d] = store(d, d % nb)
                if c + 1 < n:
                    h_i[c + 1] = idx_load(c + 1, (c + 1) % nb)

        for d in range(max(0, n - (nb - 1)), n):
            for h in h_g[d]:
                h.wait()
            h_s[d] = store(d, d % nb)
        for d in range(max(0, n - nb), n):
            h_s[d].wait()

    return gather_k


def kernel(inputs, embeddings):
    batch, hist = inputs.shape
    V, D = embeddings.shape
    if inputs.dtype != jnp.int32:
        inputs = inputs.astype(jnp.int32)
    return _build_gather(V, D, batch, hist, _CROWS, _NBUF)(embeddings, inputs)
